# baseline (speedup 1.0000x reference)
import functools

import jax
import jax.numpy as jnp
from jax import lax
from jax.experimental import pallas as pl
from jax.experimental.pallas import tpu as pltpu
from jax.experimental.pallas import tpu_sc as plsc

T = 2048
D = 768
E = 16
H = 4 * D
B = 64
NB = 50
T_PAD = NB * B
NW = 32
CHUNK = T // NW

HK = 4
CH = H // HK
NSLOT = E * HK
NBUF = 6
LOOKAHEAD = 4
G = 4
GM = G * B


def _meta_kernel(x_ref, wr_ref, br_ref, pos_ref, nblk_ref, bstart_ref):
    x = x_ref[...]
    logits = jnp.dot(x, wr_ref[...], preferred_element_type=jnp.float32)
    logits = logits + br_ref[...][None, :]

    iota_e = lax.broadcasted_iota(jnp.int32, (T, E), 1)
    mx = jnp.max(logits, axis=1, keepdims=True)
    idx = jnp.min(jnp.where(logits == mx, iota_e, E), axis=1, keepdims=True)

    oh = (iota_e == idx).astype(jnp.float32)
    rt = lax.broadcasted_iota(jnp.int32, (T, T), 0)
    ct = lax.broadcasted_iota(jnp.int32, (T, T), 1)
    tril = (ct <= rt).astype(jnp.float32)
    incl = jnp.dot(tril, oh, preferred_element_type=jnp.float32)
    rank = jnp.sum(incl * oh, axis=1, keepdims=True) - 1.0

    counts = incl[T - 1:T, :]
    nblk = jnp.floor((counts + (B - 1)) * (1.0 / B))
    re = lax.broadcasted_iota(jnp.int32, (E, E), 0)
    ce = lax.broadcasted_iota(jnp.int32, (E, E), 1)
    cum_mask = (re <= ce).astype(jnp.float32)
    cum_blk = jnp.dot(nblk, cum_mask, preferred_element_type=jnp.float32)
    bstart = cum_blk - nblk

    base = jnp.sum(oh * (bstart * B), axis=1, keepdims=True)
    pos_ref[...] = (base + rank).astype(jnp.int32).reshape(T)
    nblk_ref[...] = nblk.astype(jnp.int32).reshape(E)
    bstart_ref[...] = bstart.astype(jnp.int32).reshape(E)


def _ffn_kernel(nblk_ref, bstart_ref, xs_ref, w1_hbm, w2_hbm, b1_ref, b2_ref,
                lnw_ref, lnb_ref, out_ref, w1_buf, w2_buf, xbf_ref,
                w1_sem, w2_sem):
    def start(s):
        e = s // HK
        k = s - e * HK
        b = s % NBUF
        pltpu.make_async_copy(
            w1_hbm.at[e, :, pl.ds(k * CH, CH)], w1_buf.at[b], w1_sem.at[b]
        ).start()
        pltpu.make_async_copy(
            w2_hbm.at[e, pl.ds(k * CH, CH), :], w2_buf.at[b], w2_sem.at[b]
        ).start()

    def wait(s):
        b = s % NBUF
        pltpu.make_async_copy(
            w1_hbm.at[0, :, pl.ds(0, CH)], w1_buf.at[b], w1_sem.at[b]
        ).wait()
        pltpu.make_async_copy(
            w2_hbm.at[0, pl.ds(0, CH), :], w2_buf.at[b], w2_sem.at[b]
        ).wait()

    for p in range(LOOKAHEAD):
        start(p)

    def cast_body(i, _):
        r0 = i * 320
        xbf_ref[pl.ds(r0, 320), :] = xs_ref[pl.ds(r0, 320), :].astype(jnp.bfloat16)
        return 0

    lax.fori_loop(0, T_PAD // 320, cast_body, 0)

    def slot_body(s, _):
        @pl.when(s + LOOKAHEAD < NSLOT)
        def _():
            start(s + LOOKAHEAD)

        wait(s)
        e = s // HK
        k = s - e * HK
        b = s % NBUF
        w1c = w1_buf[b].astype(jnp.bfloat16)
        w2c = w2_buf[b].astype(jnp.bfloat16)
        b1c = b1_ref[pl.ds(e, 1), pl.ds(k * CH, CH)]

        def grp_body(j, _):
            row0 = pl.multiple_of(bstart_ref[e] * B + j * GM, B)
            valid = nblk_ref[e] * B - j * GM
            rmask = lax.broadcasted_iota(jnp.int32, (GM, 1), 0) < valid
            hc = jnp.dot(xbf_ref[pl.ds(row0, GM), :], w1c,
                         preferred_element_type=jnp.float32)
            hc = hc + b1c
            hc = 0.5 * hc * (1.0 + lax.erf(hc * (2.0 ** -0.5)))
            yp = jnp.dot(hc.astype(jnp.bfloat16), w2c,
                         preferred_element_type=jnp.float32)
            old = out_ref[pl.ds(row0, GM), :]

            @pl.when(k == 0)
            def _():
                out_ref[pl.ds(row0, GM), :] = jnp.where(rmask, yp, old)

            @pl.when((k > 0) & (k < HK - 1))
            def _():
                out_ref[pl.ds(row0, GM), :] = jnp.where(rmask, old + yp, old)

            @pl.when(k == HK - 1)
            def _():
                r = old + yp + xs_ref[pl.ds(row0, GM), :] + b2_ref[pl.ds(e, 1), :]
                mu = jnp.mean(r, axis=1, keepdims=True)
                d = r - mu
                var = jnp.mean(d * d, axis=1, keepdims=True)
                ln = (d * lax.rsqrt(var + 1e-5) * lnw_ref[...][None, :]
                      + lnb_ref[...][None, :])
                out_ref[pl.ds(row0, GM), :] = jnp.where(rmask, ln, old)

            return 0

        lax.fori_loop(0, (nblk_ref[e] + (G - 1)) // G, grp_body, 0)
        return 0

    lax.fori_loop(0, NSLOT, slot_body, 0)


@functools.cache
def _sc_kernels():
    mesh = plsc.VectorSubcoreMesh(core_axis_name="c", subcore_axis_name="s")
    scratch = [
        pltpu.VMEM((CHUNK,), jnp.int32),
        pltpu.VMEM((CHUNK, D), jnp.float32),
        pltpu.SemaphoreType.DMA,
    ]

    @functools.partial(
        pl.kernel, mesh=mesh,
        out_type=jax.ShapeDtypeStruct((T_PAD, D), jnp.float32),
        scratch_types=scratch,
    )
    def sc_scatter(x_hbm, pos_hbm, out_hbm, idx_v, rows_v, sem):
        wid = lax.axis_index("s") * 2 + lax.axis_index("c")
        base = wid * CHUNK
        pltpu.sync_copy(pos_hbm.at[pl.ds(base, CHUNK)], idx_v)
        pltpu.sync_copy(x_hbm.at[pl.ds(base, CHUNK)], rows_v)
        pltpu.async_copy(rows_v, out_hbm.at[idx_v], sem).wait()

    @functools.partial(
        pl.kernel, mesh=mesh,
        out_type=jax.ShapeDtypeStruct((T, D), jnp.float32),
        scratch_types=scratch,
    )
    def sc_gather(ys_hbm, pos_hbm, out_hbm, idx_v, rows_v, sem):
        wid = lax.axis_index("s") * 2 + lax.axis_index("c")
        base = wid * CHUNK
        pltpu.sync_copy(pos_hbm.at[pl.ds(base, CHUNK)], idx_v)
        pltpu.async_copy(ys_hbm.at[idx_v], rows_v, sem).wait()
        pltpu.sync_copy(rows_v, out_hbm.at[pl.ds(base, CHUNK)])

    return sc_scatter, sc_gather


def _ffn_call(nblk, bstart, x_sorted, W1, W2, b1, b2, ln_w, ln_b):
    return pl.pallas_call(
        _ffn_kernel,
        in_specs=[
            pl.BlockSpec(memory_space=pltpu.SMEM),
            pl.BlockSpec(memory_space=pltpu.SMEM),
            pl.BlockSpec(memory_space=pltpu.VMEM),
            pl.BlockSpec(memory_space=pl.ANY),
            pl.BlockSpec(memory_space=pl.ANY),
            pl.BlockSpec(memory_space=pltpu.VMEM),
            pl.BlockSpec(memory_space=pltpu.VMEM),
            pl.BlockSpec(memory_space=pltpu.VMEM),
            pl.BlockSpec(memory_space=pltpu.VMEM),
        ],
        out_specs=pl.BlockSpec(memory_space=pltpu.VMEM),
        out_shape=jax.ShapeDtypeStruct((T_PAD, D), jnp.float32),
        scratch_shapes=[
            pltpu.VMEM((NBUF, D, CH), jnp.float32),
            pltpu.VMEM((NBUF, CH, D), jnp.float32),
            pltpu.VMEM((T_PAD, D), jnp.bfloat16),
            pltpu.SemaphoreType.DMA((NBUF,)),
            pltpu.SemaphoreType.DMA((NBUF,)),
        ],
    )(nblk, bstart, x_sorted, W1, W2, b1, b2, ln_w, ln_b)


def kernel(x, Wr, br, W1, b1, W2, b2, ln_w, ln_b):
    pos, nblk, bstart = pl.pallas_call(
        _meta_kernel,
        out_shape=(
            jax.ShapeDtypeStruct((T,), jnp.int32),
            jax.ShapeDtypeStruct((E,), jnp.int32),
            jax.ShapeDtypeStruct((E,), jnp.int32),
        ),
    )(x, Wr, br)

    sc_scatter, sc_gather = _sc_kernels()
    x_sorted = sc_scatter(x, pos)

    y_sorted = _ffn_call(nblk, bstart, x_sorted, W1, W2, b1, b2, ln_w, ln_b)

    return sc_gather(y_sorted, pos)


# speedup vs baseline: 1.0539x; 1.0539x over previous
import functools

import jax
import jax.numpy as jnp
from jax import lax
from jax.experimental import pallas as pl
from jax.experimental.pallas import tpu as pltpu
from jax.experimental.pallas import tpu_sc as plsc

T = 2048
D = 768
E = 16
H = 4 * D
B = 64
NB = 50
T_PAD = NB * B
NW = 32
CHUNK = T // NW

HK = 2
CH = H // HK
NSLOT = E * HK
NBUF = 3
LOOKAHEAD = 2
G = 4
GM = G * B


def _meta_kernel(x_ref, wr_ref, br_ref, pos_ref, nblk_ref, bstart_ref):
    x = x_ref[...]
    logits = jnp.dot(x, wr_ref[...], preferred_element_type=jnp.float32)
    logits = logits + br_ref[...][None, :]

    iota_e = lax.broadcasted_iota(jnp.int32, (T, E), 1)
    mx = jnp.max(logits, axis=1, keepdims=True)
    idx = jnp.min(jnp.where(logits == mx, iota_e, E), axis=1, keepdims=True)

    oh = (iota_e == idx).astype(jnp.float32)
    rt = lax.broadcasted_iota(jnp.int32, (T, T), 0)
    ct = lax.broadcasted_iota(jnp.int32, (T, T), 1)
    tril = (ct <= rt).astype(jnp.float32)
    incl = jnp.dot(tril, oh, preferred_element_type=jnp.float32)
    rank = jnp.sum(incl * oh, axis=1, keepdims=True) - 1.0

    counts = incl[T - 1:T, :]
    nblk = jnp.floor((counts + (B - 1)) * (1.0 / B))
    re = lax.broadcasted_iota(jnp.int32, (E, E), 0)
    ce = lax.broadcasted_iota(jnp.int32, (E, E), 1)
    cum_mask = (re <= ce).astype(jnp.float32)
    cum_blk = jnp.dot(nblk, cum_mask, preferred_element_type=jnp.float32)
    bstart = cum_blk - nblk

    base = jnp.sum(oh * (bstart * B), axis=1, keepdims=True)
    pos_ref[...] = (base + rank).astype(jnp.int32).reshape(T)
    nblk_ref[...] = nblk.astype(jnp.int32).reshape(E)
    bstart_ref[...] = bstart.astype(jnp.int32).reshape(E)


def _ffn_kernel(nblk_ref, bstart_ref, xs_ref, w1_hbm, w2_hbm, b1_ref, b2_ref,
                lnw_ref, lnb_ref, out_ref, w1_buf, w2_buf, w1_sem, w2_sem):
    def start(s):
        e = s // HK
        k = s - e * HK
        b = s % NBUF
        pltpu.make_async_copy(
            w1_hbm.at[e, :, pl.ds(k * CH, CH)], w1_buf.at[b], w1_sem.at[b]
        ).start()
        pltpu.make_async_copy(
            w2_hbm.at[e, pl.ds(k * CH, CH), :], w2_buf.at[b], w2_sem.at[b]
        ).start()

    def wait(s):
        b = s % NBUF
        pltpu.make_async_copy(
            w1_hbm.at[0, :, pl.ds(0, CH)], w1_buf.at[b], w1_sem.at[b]
        ).wait()
        pltpu.make_async_copy(
            w2_hbm.at[0, pl.ds(0, CH), :], w2_buf.at[b], w2_sem.at[b]
        ).wait()

    for p in range(LOOKAHEAD):
        start(p)

    def slot_body(s, _):
        @pl.when(s + LOOKAHEAD < NSLOT)
        def _():
            start(s + LOOKAHEAD)

        wait(s)
        e = s // HK
        k = s - e * HK
        b = s % NBUF
        w1c = w1_buf[b].astype(jnp.bfloat16)
        w2c = w2_buf[b].astype(jnp.bfloat16)
        b1c = b1_ref[pl.ds(e, 1), pl.ds(k * CH, CH)]

        def grp_body(j, _):
            row0 = pl.multiple_of(bstart_ref[e] * B + j * GM, B)
            valid = nblk_ref[e] * B - j * GM
            rmask = lax.broadcasted_iota(jnp.int32, (GM, 1), 0) < valid
            xb = xs_ref[pl.ds(row0, GM), :]
            hc = jnp.dot(xb.astype(jnp.bfloat16), w1c,
                         preferred_element_type=jnp.float32)
            hc = hc + b1c
            hc = 0.5 * hc * (1.0 + lax.erf(hc * (2.0 ** -0.5)))
            yp = jnp.dot(hc.astype(jnp.bfloat16), w2c,
                         preferred_element_type=jnp.float32)
            old = out_ref[pl.ds(row0, GM), :]

            @pl.when(k == 0)
            def _():
                out_ref[pl.ds(row0, GM), :] = jnp.where(rmask, yp, old)

            @pl.when((k > 0) & (k < HK - 1))
            def _():
                out_ref[pl.ds(row0, GM), :] = jnp.where(rmask, old + yp, old)

            @pl.when(k == HK - 1)
            def _():
                r = old + yp + xb + b2_ref[pl.ds(e, 1), :]
                mu = jnp.mean(r, axis=1, keepdims=True)
                d = r - mu
                var = jnp.mean(d * d, axis=1, keepdims=True)
                ln = (d * lax.rsqrt(var + 1e-5) * lnw_ref[...][None, :]
                      + lnb_ref[...][None, :])
                out_ref[pl.ds(row0, GM), :] = jnp.where(rmask, ln, old)

            return 0

        lax.fori_loop(0, (nblk_ref[e] + (G - 1)) // G, grp_body, 0)
        return 0

    lax.fori_loop(0, NSLOT, slot_body, 0)


@functools.cache
def _sc_kernels():
    mesh = plsc.VectorSubcoreMesh(core_axis_name="c", subcore_axis_name="s")
    scratch = [
        pltpu.VMEM((CHUNK,), jnp.int32),
        pltpu.VMEM((CHUNK, D), jnp.float32),
        pltpu.SemaphoreType.DMA,
    ]

    @functools.partial(
        pl.kernel, mesh=mesh,
        out_type=jax.ShapeDtypeStruct((T_PAD, D), jnp.float32),
        scratch_types=scratch,
    )
    def sc_scatter(x_hbm, pos_hbm, out_hbm, idx_v, rows_v, sem):
        wid = lax.axis_index("s") * 2 + lax.axis_index("c")
        base = wid * CHUNK
        pltpu.sync_copy(pos_hbm.at[pl.ds(base, CHUNK)], idx_v)
        pltpu.sync_copy(x_hbm.at[pl.ds(base, CHUNK)], rows_v)
        pltpu.async_copy(rows_v, out_hbm.at[idx_v], sem).wait()

    @functools.partial(
        pl.kernel, mesh=mesh,
        out_type=jax.ShapeDtypeStruct((T, D), jnp.float32),
        scratch_types=scratch,
    )
    def sc_gather(ys_hbm, pos_hbm, out_hbm, idx_v, rows_v, sem):
        wid = lax.axis_index("s") * 2 + lax.axis_index("c")
        base = wid * CHUNK
        pltpu.sync_copy(pos_hbm.at[pl.ds(base, CHUNK)], idx_v)
        pltpu.async_copy(ys_hbm.at[idx_v], rows_v, sem).wait()
        pltpu.sync_copy(rows_v, out_hbm.at[pl.ds(base, CHUNK)])

    return sc_scatter, sc_gather


def _ffn_call(nblk, bstart, x_sorted, W1, W2, b1, b2, ln_w, ln_b):
    return pl.pallas_call(
        _ffn_kernel,
        in_specs=[
            pl.BlockSpec(memory_space=pltpu.SMEM),
            pl.BlockSpec(memory_space=pltpu.SMEM),
            pl.BlockSpec(memory_space=pltpu.VMEM),
            pl.BlockSpec(memory_space=pl.ANY),
            pl.BlockSpec(memory_space=pl.ANY),
            pl.BlockSpec(memory_space=pltpu.VMEM),
            pl.BlockSpec(memory_space=pltpu.VMEM),
            pl.BlockSpec(memory_space=pltpu.VMEM),
            pl.BlockSpec(memory_space=pltpu.VMEM),
        ],
        out_specs=pl.BlockSpec(memory_space=pltpu.VMEM),
        out_shape=jax.ShapeDtypeStruct((T_PAD, D), jnp.float32),
        scratch_shapes=[
            pltpu.VMEM((NBUF, D, CH), jnp.float32),
            pltpu.VMEM((NBUF, CH, D), jnp.float32),
            pltpu.SemaphoreType.DMA((NBUF,)),
            pltpu.SemaphoreType.DMA((NBUF,)),
        ],
    )(nblk, bstart, x_sorted, W1, W2, b1, b2, ln_w, ln_b)


def kernel(x, Wr, br, W1, b1, W2, b2, ln_w, ln_b):
    pos, nblk, bstart = pl.pallas_call(
        _meta_kernel,
        out_shape=(
            jax.ShapeDtypeStruct((T,), jnp.int32),
            jax.ShapeDtypeStruct((E,), jnp.int32),
            jax.ShapeDtypeStruct((E,), jnp.int32),
        ),
    )(x, Wr, br)

    sc_scatter, sc_gather = _sc_kernels()
    x_sorted = sc_scatter(x, pos)

    y_sorted = _ffn_call(nblk, bstart, x_sorted, W1, W2, b1, b2, ln_w, ln_b)

    return sc_gather(y_sorted, pos)
